# baseline (device time: 19005 ns/iter reference)
import jax
import jax.numpy as jnp
from jax import lax
from jax.experimental import pallas as pl
from jax.experimental.pallas import tpu as pltpu

NBLK = 8


def kernel(x, dy, gamma):
    del gamma
    m, d = x.shape
    half = m // 2
    mb = half // NBLK

    def body(
        x_hbm, dy_hbm, out_ref,
        xbuf, dybuf, comm_ref,
        load_sems, send_sems, recv_sems,
    ):
        my_x = lax.axis_index("x")
        my_y = lax.axis_index("y")
        peers = (
            (my_x, 1 - my_y),
            (1 - my_x, my_y),
            (1 - my_x, 1 - my_y),
        )

        copies = []
        for i in range(NBLK):
            cx = pltpu.make_async_copy(
                x_hbm.at[pl.ds(my_x * half + i * mb, mb)],
                xbuf.at[i], load_sems.at[i, 0],
            )
            cy = pltpu.make_async_copy(
                dy_hbm.at[pl.ds(my_x * half + i * mb, mb)],
                dybuf.at[i], load_sems.at[i, 1],
            )
            cx.start()
            cy.start()
            copies.append((cx, cy))

        barrier = pltpu.get_barrier_semaphore()
        for nbr in peers:
            pl.semaphore_signal(
                barrier, inc=1, device_id=nbr,
                device_id_type=pl.DeviceIdType.MESH,
            )

        dgamma = jnp.zeros((1, d), jnp.float32)
        dbeta = jnp.zeros((1, d), jnp.float32)
        for i in range(NBLK):
            cx, cy = copies[i]
            cx.wait()
            cy.wait()
            xb = xbuf[i]
            dyb = dybuf[i]
            mu = jnp.mean(xb, axis=1, keepdims=True)
            xc = xb - mu
            var = jnp.mean(xc * xc, axis=1, keepdims=True)
            xhat = xc * lax.rsqrt(var + 1e-5)
            dgamma += jnp.sum(dyb * xhat, axis=0, keepdims=True)
            dbeta += jnp.sum(dyb, axis=0, keepdims=True)

        out_ref[...] = jnp.concatenate([dgamma, dbeta], axis=0)

        pl.semaphore_wait(barrier, 3)

        rdmas = []
        for k, nbr in enumerate(peers):
            rdma = pltpu.make_async_remote_copy(
                src_ref=out_ref,
                dst_ref=comm_ref.at[k],
                send_sem=send_sems.at[k],
                recv_sem=recv_sems.at[k],
                device_id=nbr,
                device_id_type=pl.DeviceIdType.MESH,
            )
            rdma.start()
            rdmas.append(rdma)
        for rdma in rdmas:
            rdma.wait()
        out_ref[...] += comm_ref[0] + comm_ref[1] + comm_ref[2]

    return pl.pallas_call(
        body,
        out_shape=jax.ShapeDtypeStruct((2, d), jnp.float32),
        in_specs=[
            pl.BlockSpec(memory_space=pltpu.MemorySpace.HBM),
            pl.BlockSpec(memory_space=pltpu.MemorySpace.HBM),
        ],
        out_specs=pl.BlockSpec(memory_space=pltpu.MemorySpace.VMEM),
        scratch_shapes=[
            pltpu.VMEM((NBLK, mb, d), jnp.float32),
            pltpu.VMEM((NBLK, mb, d), jnp.float32),
            pltpu.VMEM((3, 2, d), jnp.float32),
            pltpu.SemaphoreType.DMA((NBLK, 2)),
            pltpu.SemaphoreType.DMA((3,)),
            pltpu.SemaphoreType.DMA((3,)),
        ],
        compiler_params=pltpu.CompilerParams(
            collective_id=0,
            vmem_limit_bytes=60 * 1024 * 1024,
        ),
    )(x, dy)
